# Initial kernel scaffold; baseline (speedup 1.0000x reference)
import jax
import jax.numpy as jnp
from jax.experimental import pallas as pl


def kernel(nodes_u, nodes_v, global_protos, inter_nums, u_emb_w, v_emb_w, u_rev_w, v_rev_w, W1, b1, W2, b2, W3, b3, Wc, bc, g1, be1, g2, be2, g3, be3):
    raise NotImplementedError("write your pallas kernel here")


# speedup vs baseline: 1.6593x; 1.6593x over previous
import functools

import jax
import jax.numpy as jnp
from jax import lax
from jax.experimental import pallas as pl
from jax.experimental.pallas import tpu as pltpu
from jax.experimental.pallas import tpu_sc as plsc

_B = 16384
_D = 128
_NC = 2
_NS = 16
_NW = _NC * _NS
_BPW = _B // _NW


_HC = _BPW // 4


def _sc_gather_body(u_emb, v_emb, u_rev, v_rev, idx_u, idx_v, neg_flat, loc_idx,
                    u_id_out, v_id_out, u_rev_out, v_rev_out, pot_out,
                    idx_buf, idx_h, loc, stage, rows, acc, sem):
    c = lax.axis_index("c")
    s = lax.axis_index("s")
    wid = s * _NC + c
    base = wid * _BPW
    sb = s * _HC
    pltpu.sync_copy(loc_idx.at[pl.ds(sb, _HC)], loc)
    for h in range(4):
        hb = base + h * _HC
        pltpu.sync_copy(neg_flat.at[pl.ds(hb, _HC)], idx_h)
        pltpu.async_copy(v_emb.at[idx_h], stage, sem).wait()
        pltpu.sync_copy(stage, acc.at[pl.ds(sb, _HC)])
        for j in range(1, 4):
            pltpu.sync_copy(neg_flat.at[pl.ds(j * _B + hb, _HC)], idx_h)
            pltpu.async_copy(v_emb.at[idx_h], stage, sem).wait()
            pltpu.sync_copy(stage, acc.at[loc], add=True)
        pltpu.sync_copy(acc.at[pl.ds(sb, _HC)], stage)
        pltpu.sync_copy(stage, pot_out.at[pl.ds(hb, _HC)])
    for table, isrc, out in ((u_emb, idx_u, u_id_out),
                             (v_emb, idx_v, v_id_out),
                             (u_rev, idx_u, u_rev_out),
                             (v_rev, idx_v, v_rev_out)):
        pltpu.sync_copy(isrc.at[pl.ds(base, _BPW)], idx_buf)
        pltpu.async_copy(table.at[idx_buf], rows, sem).wait()
        pltpu.sync_copy(rows, out.at[pl.ds(base, _BPW)])


@functools.lru_cache(maxsize=None)
def _get_sc_call():
    mesh = plsc.VectorSubcoreMesh(
        core_axis_name="c", subcore_axis_name="s",
        num_cores=_NC, num_subcores=_NS)
    return pl.kernel(
        _sc_gather_body,
        out_type=[jax.ShapeDtypeStruct((_B, _D), jnp.float32)] * 5,
        mesh=mesh,
        scratch_types=[
            pltpu.VMEM((_BPW,), jnp.int32),
            pltpu.VMEM((_HC,), jnp.int32),
            pltpu.VMEM((_HC,), jnp.int32),
            pltpu.VMEM((_HC, _D), jnp.float32),
            pltpu.VMEM((_BPW, _D), jnp.float32),
            pltpu.VMEM_SHARED((_NS * _HC, _D), jnp.float32),
            pltpu.SemaphoreType.DMA,
        ],
    )


def _bn_train(x, g, b):
    mu = jnp.mean(x, axis=0, keepdims=True)
    xc = x - mu
    var = jnp.mean(xc * xc, axis=0, keepdims=True)
    return g * (xc * lax.rsqrt(var + 1e-5)) + b


def _tc_body(u_id, v_raw, pot_s, delta, W1, W2, W3, Wc,
             b1, b2, b3, bc, g1, be1, g2, be2, g3, be3,
             pred_out, vmix_out):
    dn = (((1,), (1,)), ((), ()))
    f32 = jnp.float32
    d = delta[...]
    pot = pot_s[...] * 0.25
    vmix = d * v_raw[...] + (1.0 - d) * pot
    vmix_out[...] = vmix
    w1 = W1[...]
    a1 = (lax.dot_general(u_id[...], w1[:, :_D], dn, preferred_element_type=f32)
          + lax.dot_general(vmix, w1[:, _D:], dn, preferred_element_type=f32))
    z1 = jnp.maximum(a1 + b1[...], 0.0)
    h1 = _bn_train(z1, g1[...], be1[...])
    z2 = jnp.maximum(
        lax.dot_general(h1, W2[...], dn, preferred_element_type=f32) + b2[...], 0.0)
    h2 = _bn_train(z2, g2[...], be2[...])
    z3 = jnp.maximum(
        lax.dot_general(h2, W3[...], dn, preferred_element_type=f32) + b3[...], 0.0)
    h3 = _bn_train(z3, g3[...], be3[...])
    logit = jnp.sum(h3 * Wc[...], axis=1, keepdims=True) + bc[...]
    pred_out[...] = 1.0 / (1.0 + jnp.exp(-logit))


_tc_call = pl.pallas_call(
    _tc_body,
    out_shape=(jax.ShapeDtypeStruct((_B, 1), jnp.float32),
               jax.ShapeDtypeStruct((_B, _D), jnp.float32)),
)


def kernel(nodes_u, nodes_v, global_protos, inter_nums, u_emb_w, v_emb_w,
           u_rev_w, v_rev_w, W1, b1, W2, b2, W3, b3, Wc, bc,
           g1, be1, g2, be2, g3, be3):
    nB = nodes_u.shape[0]
    kk = jax.random.key(42)
    k1, k2 = jax.random.split(kk)
    neg_items = jax.random.randint(k1, (nB, 4), 0, v_emb_w.shape[0])
    delta = jnp.clip(
        jax.random.normal(k2, (nB, u_emb_w.shape[1]), jnp.float32) * 0.1 + 0.5,
        0.0, 1.0)
    neg_flat = jnp.asarray(neg_items).T.reshape(-1).astype(jnp.int32)
    loc_idx = jnp.arange(_NS * _HC, dtype=jnp.int32)
    idx_u = nodes_u.astype(jnp.int32)
    idx_v = nodes_v.astype(jnp.int32)

    u_id, v_id_raw, u_rev, v_rev, pot_sum = _get_sc_call()(
        u_emb_w, v_emb_w, u_rev_w, v_rev_w, idx_u, idx_v, neg_flat, loc_idx)

    r = lambda a: a.reshape(1, -1)
    pred, vmix = _tc_call(
        u_id, v_id_raw, pot_sum, delta, W1, W2, W3, Wc,
        r(b1), r(b2), r(b3), bc.reshape(1, 1),
        r(g1), r(be1), r(g2), r(be2), r(g3), r(be3))
    return (pred.reshape(nB), u_id, vmix, u_rev, v_rev)


# speedup vs baseline: 1.7264x; 1.0404x over previous
import functools

import jax
import jax.numpy as jnp
from jax import lax
from jax.experimental import pallas as pl
from jax.experimental.pallas import tpu as pltpu
from jax.experimental.pallas import tpu_sc as plsc

_B = 16384
_D = 128
_NC = 2
_NS = 16
_NW = _NC * _NS
_BPW = _B // _NW


_GC = 256
_PC = _GC // 4


def _pool4(g, p):
    def body_fn(r, carry):
        r4 = 4 * r
        for sl16 in range(_D // 16):
            sl = pl.ds(sl16 * 16, 16)
            p[r, sl] = g[r4, sl] + g[r4 + 1, sl] + g[r4 + 2, sl] + g[r4 + 3, sl]
        return carry
    lax.fori_loop(0, _PC, body_fn, 0)


def _sc_gather_body(u_emb, v_emb, u_rev, v_rev, idx_u, idx_v, neg_flat,
                    u_id_out, v_id_out, u_rev_out, v_rev_out, pot_out,
                    idx_u_b, idx_v_b, idx_n, gA, gB, pA, pB,
                    sg0, sg1, sw0, sw1):
    c = lax.axis_index("c")
    s = lax.axis_index("s")
    wid = s * _NC + c
    base = wid * _BPW
    pltpu.sync_copy(idx_u.at[pl.ds(base, _BPW)], idx_u_b)
    pltpu.sync_copy(idx_v.at[pl.ds(base, _BPW)], idx_v_b)
    pltpu.sync_copy(neg_flat.at[pl.ds(4 * base, 4 * _BPW)], idx_n)

    mains = [(t, h) for t in ((u_emb, idx_u_b, u_id_out),
                              (v_emb, idx_v_b, v_id_out),
                              (u_rev, idx_u_b, u_rev_out),
                              (v_rev, idx_v_b, v_rev_out)) for h in range(2)]
    jobs = []
    for i in range(8):
        jobs.append(("neg", i))
        jobs.append(("main", mains[i]))

    def start_gather(k, buf, sem):
        kind, d = jobs[k]
        if kind == "neg":
            i = d
            return pltpu.async_copy(
                v_emb.at[idx_n.at[pl.ds(i * _GC, _GC)]], buf, sem)
        (table, ib, _), h = d
        return pltpu.async_copy(
            table.at[ib.at[pl.ds(h * _GC, _GC)]], buf, sem)

    gbuf = {"neg": gA, "main": gB}
    gsem = {"neg": sg0, "main": sg1}
    wsem = {"neg": sw0, "main": sw1}
    pbuf = [pA, pB]
    wpend = {"neg": None, "main": None}
    ppend = [None, None]
    hg = {}
    hg[jobs[0][0]] = start_gather(0, gbuf[jobs[0][0]], gsem[jobs[0][0]])
    for k in range(16):
        kind, d = jobs[k]
        if k + 1 < 16:
            nkind = jobs[k + 1][0]
            if wpend[nkind] is not None:
                wpend[nkind].wait()
                wpend[nkind] = None
            hg[nkind] = start_gather(k + 1, gbuf[nkind], gsem[nkind])
        hg[kind].wait()
        if kind == "neg":
            i = d
            pb = pbuf[i % 2]
            if ppend[i % 2] is not None:
                ppend[i % 2].wait()
                ppend[i % 2] = None
            _pool4(gA, pb)
            ppend[i % 2] = pltpu.async_copy(
                pb, pot_out.at[pl.ds(base + i * _PC, _PC)], wsem["neg"])
        else:
            (_, _, out), h = d
            wpend["main"] = pltpu.async_copy(
                gB, out.at[pl.ds(base + h * _GC, _GC)], wsem["main"])
    for hnd in (wpend["neg"], wpend["main"], ppend[0], ppend[1]):
        if hnd is not None:
            hnd.wait()


@functools.lru_cache(maxsize=None)
def _get_sc_call():
    mesh = plsc.VectorSubcoreMesh(
        core_axis_name="c", subcore_axis_name="s",
        num_cores=_NC, num_subcores=_NS)
    return pl.kernel(
        _sc_gather_body,
        out_type=[jax.ShapeDtypeStruct((_B, _D), jnp.float32)] * 5,
        mesh=mesh,
        scratch_types=[
            pltpu.VMEM((_BPW,), jnp.int32),
            pltpu.VMEM((_BPW,), jnp.int32),
            pltpu.VMEM((4 * _BPW,), jnp.int32),
            pltpu.VMEM((_GC, _D), jnp.float32),
            pltpu.VMEM((_GC, _D), jnp.float32),
            pltpu.VMEM((_PC, _D), jnp.float32),
            pltpu.VMEM((_PC, _D), jnp.float32),
            pltpu.SemaphoreType.DMA,
            pltpu.SemaphoreType.DMA,
            pltpu.SemaphoreType.DMA,
            pltpu.SemaphoreType.DMA,
        ],
    )


def _bn_train(x, g, b):
    mu = jnp.mean(x, axis=0, keepdims=True)
    xc = x - mu
    var = jnp.mean(xc * xc, axis=0, keepdims=True)
    return g * (xc * lax.rsqrt(var + 1e-5)) + b


def _tc_body(u_id, v_raw, pot_s, delta, W1, W2, W3, Wc,
             b1, b2, b3, bc, g1, be1, g2, be2, g3, be3,
             pred_out, vmix_out):
    dn = (((1,), (1,)), ((), ()))
    f32 = jnp.float32
    d = delta[...]
    pot = pot_s[...] * 0.25
    vmix = d * v_raw[...] + (1.0 - d) * pot
    vmix_out[...] = vmix
    w1 = W1[...]
    a1 = (lax.dot_general(u_id[...], w1[:, :_D], dn, preferred_element_type=f32)
          + lax.dot_general(vmix, w1[:, _D:], dn, preferred_element_type=f32))
    z1 = jnp.maximum(a1 + b1[...], 0.0)
    h1 = _bn_train(z1, g1[...], be1[...])
    z2 = jnp.maximum(
        lax.dot_general(h1, W2[...], dn, preferred_element_type=f32) + b2[...], 0.0)
    h2 = _bn_train(z2, g2[...], be2[...])
    z3 = jnp.maximum(
        lax.dot_general(h2, W3[...], dn, preferred_element_type=f32) + b3[...], 0.0)
    h3 = _bn_train(z3, g3[...], be3[...])
    logit = jnp.sum(h3 * Wc[...], axis=1, keepdims=True) + bc[...]
    pred_out[...] = 1.0 / (1.0 + jnp.exp(-logit))


_tc_call = pl.pallas_call(
    _tc_body,
    out_shape=(jax.ShapeDtypeStruct((_B, 1), jnp.float32),
               jax.ShapeDtypeStruct((_B, _D), jnp.float32)),
)


def kernel(nodes_u, nodes_v, global_protos, inter_nums, u_emb_w, v_emb_w,
           u_rev_w, v_rev_w, W1, b1, W2, b2, W3, b3, Wc, bc,
           g1, be1, g2, be2, g3, be3):
    nB = nodes_u.shape[0]
    kk = jax.random.key(42)
    k1, k2 = jax.random.split(kk)
    neg_items = jax.random.randint(k1, (nB, 4), 0, v_emb_w.shape[0])
    delta = jnp.clip(
        jax.random.normal(k2, (nB, u_emb_w.shape[1]), jnp.float32) * 0.1 + 0.5,
        0.0, 1.0)
    neg_flat = jnp.asarray(neg_items).reshape(-1).astype(jnp.int32)
    idx_u = nodes_u.astype(jnp.int32)
    idx_v = nodes_v.astype(jnp.int32)

    u_id, v_id_raw, u_rev, v_rev, pot_sum = _get_sc_call()(
        u_emb_w, v_emb_w, u_rev_w, v_rev_w, idx_u, idx_v, neg_flat)

    r = lambda a: a.reshape(1, -1)
    pred, vmix = _tc_call(
        u_id, v_id_raw, pot_sum, delta, W1, W2, W3, Wc,
        r(b1), r(b2), r(b3), bc.reshape(1, 1),
        r(g1), r(be1), r(g2), r(be2), r(g3), r(be3))
    return (pred.reshape(nB), u_id, vmix, u_rev, v_rev)
